# Initial kernel scaffold; baseline (speedup 1.0000x reference)
import functools

import jax
import jax.numpy as jnp
from jax import lax
from jax.experimental import pallas as pl
from jax.experimental.pallas import tpu as pltpu
from jax.experimental.pallas import tpu_sc as plsc

N = 10000
T = 4
D = 128
E = 160000

NC = 2
NS = 16
NTILES = NC * NS
EPT = E // NTILES
CH = 128
NCH = -(-EPT // CH)
EPT_PAD = NCH * CH
RPT = 632
NP = NS * RPT
DUMP = N
DEGW = 16
ZR = RPT // 4


def _mesh():
    return plsc.VectorSubcoreMesh(core_axis_name="c", subcore_axis_name="s")


def _sc_degree(dst_t):
    @functools.partial(
        pl.kernel,
        out_type=jax.ShapeDtypeStruct((NC, NP, DEGW), jnp.float32),
        mesh=_mesh(),
        scratch_types=[
            pltpu.VMEM_SHARED((NP, DEGW), jnp.float32),
            pltpu.VMEM((NCH, CH), jnp.int32),
            pltpu.VMEM((CH, DEGW), jnp.float32),
            pltpu.VMEM((RPT, DEGW), jnp.float32),
        ],
    )
    def deg_kernel(dst_hbm, out_hbm, acc, dstv, onesv, zerov):
        c = lax.axis_index("c")
        s = lax.axis_index("s")
        wid = c * NS + s

        def fill_ones(i, carry):
            onesv[i, :] = jnp.ones((DEGW,), jnp.float32)
            return carry

        lax.fori_loop(0, CH, fill_ones, 0)

        def fill_zero(i, carry):
            zerov[i, :] = jnp.zeros((DEGW,), jnp.float32)
            return carry

        lax.fori_loop(0, RPT, fill_zero, 0)
        pltpu.sync_copy(zerov, acc.at[pl.ds(s * RPT, RPT)])
        pltpu.sync_copy(dst_hbm.at[wid], dstv)
        plsc.subcore_barrier()

        def chunk(j, carry):
            pltpu.sync_copy(onesv, acc.at[dstv.at[j]], add=True)
            return carry

        lax.fori_loop(0, NCH, chunk, 0)
        plsc.subcore_barrier()
        pltpu.sync_copy(acc.at[pl.ds(s * RPT, RPT)],
                        out_hbm.at[c, pl.ds(s * RPT, RPT)])

    return deg_kernel(dst_t)


def _sc_propagate(g, src_t, dst_t):
    @functools.partial(
        pl.kernel,
        out_type=jax.ShapeDtypeStruct((NC, NP, D), jnp.float32),
        mesh=_mesh(),
        scratch_types=[
            pltpu.VMEM_SHARED((NP, D), jnp.float32),
            pltpu.VMEM((NCH, CH), jnp.int32),
            pltpu.VMEM((NCH, CH), jnp.int32),
            pltpu.VMEM((CH, D), jnp.float32),
            pltpu.VMEM((ZR, D), jnp.float32),
            pltpu.SemaphoreType.DMA,
        ],
    )
    def prop_kernel(g_hbm, src_hbm, dst_hbm, out_hbm,
                    acc, srcv, dstv, buf, zerov, sem):
        c = lax.axis_index("c")
        s = lax.axis_index("s")
        wid = c * NS + s

        def fill_zero(q, carry):
            i = q // (D // 16)
            k = q % (D // 16)
            zerov[i, pl.ds(k * 16, 16)] = jnp.zeros((16,), jnp.float32)
            return carry

        lax.fori_loop(0, ZR * (D // 16), fill_zero, 0)

        def zero_copy(k, carry):
            pltpu.sync_copy(zerov, acc.at[pl.ds(s * RPT + k * ZR, ZR)])
            return carry

        lax.fori_loop(0, RPT // ZR, zero_copy, 0)
        pltpu.sync_copy(src_hbm.at[wid], srcv)
        pltpu.sync_copy(dst_hbm.at[wid], dstv)
        plsc.subcore_barrier()

        def chunk(j, carry):
            pltpu.async_copy(g_hbm.at[srcv.at[j]], buf, sem).wait()
            pltpu.sync_copy(buf, acc.at[dstv.at[j]], add=True)
            return carry

        lax.fori_loop(0, NCH, chunk, 0)
        plsc.subcore_barrier()
        pltpu.sync_copy(acc.at[pl.ds(s * RPT, RPT)],
                        out_hbm.at[c, pl.ds(s * RPT, RPT)])

    return prop_kernel(g, src_t, dst_t)


def _tc_dis(degp):
    B = NP // 8

    def body(p_ref, dis_ref):
        deg = p_ref[0, :, 0:1] + p_ref[1, :, 0:1] + 1.0
        dis_ref[...] = lax.rsqrt(deg)

    return pl.pallas_call(
        body,
        grid=(NP // B,),
        in_specs=[pl.BlockSpec((2, B, DEGW), lambda i: (0, i, 0))],
        out_specs=pl.BlockSpec((B, 1), lambda i: (i, 0)),
        out_shape=jax.ShapeDtypeStruct((NP, 1), jnp.float32),
    )(degp)


def _tc_mm1(x_pad, W1, dis):
    B = RPT

    def body(x_ref, w_ref, dis_ref, out_ref):
        g = jnp.dot(x_ref[:, 0, :], w_ref[...],
                    preferred_element_type=jnp.float32)
        out_ref[0] = g * dis_ref[...]

    return pl.pallas_call(
        body,
        grid=(T, NP // B),
        in_specs=[
            pl.BlockSpec((B, 1, D), lambda t, i: (i, t, 0)),
            pl.BlockSpec((D, D), lambda t, i: (0, 0)),
            pl.BlockSpec((B, 1), lambda t, i: (i, 0)),
        ],
        out_specs=pl.BlockSpec((1, B, D), lambda t, i: (t, i, 0)),
        out_shape=jax.ShapeDtypeStruct((T, NP, D), jnp.float32),
    )(x_pad, W1, dis)


def _tc_comb(part, g1, dis, b, W2, t):
    B = RPT

    def body(p_ref, g_ref, dis_ref, b_ref, w_ref, out_ref):
        ssum = p_ref[0] + p_ref[1] + g_ref[0]
        z = jnp.maximum(ssum * dis_ref[...] + b_ref[...], 0.0)
        h2 = jnp.dot(z, w_ref[...], preferred_element_type=jnp.float32)
        out_ref[...] = h2 * dis_ref[...]

    return pl.pallas_call(
        body,
        grid=(NP // B,),
        in_specs=[
            pl.BlockSpec((2, B, D), lambda i: (0, i, 0)),
            pl.BlockSpec((1, B, D), lambda i, _t=t: (_t, i, 0)),
            pl.BlockSpec((B, 1), lambda i: (i, 0)),
            pl.BlockSpec((1, D), lambda i: (0, 0)),
            pl.BlockSpec((D, D), lambda i: (0, 0)),
        ],
        out_specs=pl.BlockSpec((B, D), lambda i: (i, 0)),
        out_shape=jax.ShapeDtypeStruct((NP, D), jnp.float32),
    )(part, g1, dis, b, W2)


def _tc_final(part, g2, dis, b):
    B = RPT

    def body(p_ref, g_ref, dis_ref, b_ref, out_ref):
        ssum = p_ref[0] + p_ref[1] + g_ref[...]
        out_ref[...] = jnp.maximum(ssum * dis_ref[...] + b_ref[...], 0.0)

    return pl.pallas_call(
        body,
        grid=(NP // B,),
        in_specs=[
            pl.BlockSpec((2, B, D), lambda i: (0, i, 0)),
            pl.BlockSpec((B, D), lambda i: (i, 0)),
            pl.BlockSpec((B, 1), lambda i: (i, 0)),
            pl.BlockSpec((1, D), lambda i: (0, 0)),
        ],
        out_specs=pl.BlockSpec((B, D), lambda i: (i, 0)),
        out_shape=jax.ShapeDtypeStruct((NP, D), jnp.float32),
    )(part, g2, dis, b)


def kernel(x, edge_index, W1, b1, W2, b2):
    src = edge_index[0].astype(jnp.int32)
    dst = edge_index[1].astype(jnp.int32)
    src_t = jnp.pad(src.reshape(NTILES, EPT),
                    ((0, 0), (0, EPT_PAD - EPT))).reshape(NTILES, NCH, CH)
    dst_t = jnp.pad(dst.reshape(NTILES, EPT),
                    ((0, 0), (0, EPT_PAD - EPT)),
                    constant_values=DUMP).reshape(NTILES, NCH, CH)
    x_pad = jnp.pad(x.astype(jnp.float32), ((0, NP - N), (0, 0), (0, 0)))
    W1 = W1.astype(jnp.float32)
    W2 = W2.astype(jnp.float32)
    b1 = b1.astype(jnp.float32).reshape(1, D)
    b2 = b2.astype(jnp.float32).reshape(1, D)

    degp = _sc_degree(dst_t)
    dis = _tc_dis(degp)
    g1 = _tc_mm1(x_pad, W1, dis)

    outs = []
    for t in range(T):
        p1 = _sc_propagate(g1[t], src_t, dst_t)
        g2 = _tc_comb(p1, g1, dis, b1, W2, t)
        p2 = _sc_propagate(g2, src_t, dst_t)
        outs.append(_tc_final(p2, g2, dis, b2)[:N])
    return jnp.stack(outs, axis=1)


# speedup vs baseline: 4.9999x; 4.9999x over previous
import functools

import jax
import jax.numpy as jnp
from jax import lax
from jax.experimental import pallas as pl
from jax.experimental.pallas import tpu as pltpu
from jax.experimental.pallas import tpu_sc as plsc

N = 10000
T = 4
D = 128
E = 160000

NC = 2
NS = 16
NTILES = NC * NS
EPT = E // NTILES
CH = 128
NCH = -(-EPT // CH)
EPT_PAD = NCH * CH
RPT = 632
NP = NS * RPT
DUMP = N
DEGW = 16
ZR = RPT // 4


def _mesh():
    return plsc.VectorSubcoreMesh(core_axis_name="c", subcore_axis_name="s")


def _i32(v):
    return jnp.int32(v)


def _fori(n, body):
    lax.fori_loop(jnp.int32(0), jnp.int32(n), body, 0)


def _sc_degree(dst_t):
    @functools.partial(
        pl.kernel,
        out_type=jax.ShapeDtypeStruct((NC, NP, DEGW), jnp.float32),
        mesh=_mesh(),
        scratch_types=[
            pltpu.VMEM_SHARED((NP, DEGW), jnp.float32),
            pltpu.VMEM((NCH, CH), jnp.int32),
            pltpu.VMEM((CH, DEGW), jnp.float32),
            pltpu.VMEM((RPT, DEGW), jnp.float32),
        ],
    )
    def deg_kernel(dst_hbm, out_hbm, acc, dstv, onesv, zerov):
        c = lax.axis_index("c")
        s = lax.axis_index("s")
        wid = c * NS + s

        def fill_ones(i, carry):
            onesv[i, :] = jnp.ones((DEGW,), jnp.float32)
            return carry

        _fori(CH, fill_ones)

        def fill_zero(i, carry):
            zerov[i, :] = jnp.zeros((DEGW,), jnp.float32)
            return carry

        _fori(RPT, fill_zero)
        pltpu.sync_copy(zerov, acc.at[pl.ds(s * RPT, RPT)])
        pltpu.sync_copy(dst_hbm.at[wid], dstv)
        plsc.subcore_barrier()

        def chunk(j, carry):
            pltpu.sync_copy(onesv, acc.at[dstv.at[j]], add=True)
            return carry

        _fori(NCH, chunk)
        plsc.subcore_barrier()
        pltpu.sync_copy(acc.at[pl.ds(s * RPT, RPT)],
                        out_hbm.at[c, pl.ds(s * RPT, RPT)])

    return deg_kernel(dst_t)


def _sc_propagate(g, src_t, dst_t):
    @functools.partial(
        pl.kernel,
        out_type=jax.ShapeDtypeStruct((NC, NP, D), jnp.float32),
        mesh=_mesh(),
        scratch_types=[
            pltpu.VMEM_SHARED((NP, D), jnp.float32),
            pltpu.VMEM((NCH, CH), jnp.int32),
            pltpu.VMEM((NCH, CH), jnp.int32),
            pltpu.VMEM((2, CH, D), jnp.float32),
            pltpu.SemaphoreType.DMA,
            pltpu.SemaphoreType.DMA,
        ],
    )
    def prop_kernel(g_hbm, src_hbm, dst_hbm, out_hbm,
                    acc, srcv, dstv, buf2, gsem, ssem):
        c = lax.axis_index("c")
        s = lax.axis_index("s")
        wid = c * NS + s

        def fill_zero(q, carry):
            i = q // (D // 16)
            k = q % (D // 16)
            buf2[1, i, pl.ds(k * 16, 16)] = jnp.zeros((16,), jnp.float32)
            return carry

        _fori(CH * (D // 16), fill_zero)

        def zero_copy(k, carry):
            pltpu.sync_copy(buf2.at[_i32(1)],
                            acc.at[pl.ds(s * RPT + k * CH, CH)])
            return carry

        _fori(RPT // CH, zero_copy)
        pltpu.sync_copy(buf2.at[_i32(1), pl.ds(0, RPT % CH)],
                        acc.at[pl.ds(s * RPT + (RPT // CH) * CH, RPT % CH)])
        pltpu.sync_copy(src_hbm.at[wid], srcv)
        pltpu.sync_copy(dst_hbm.at[wid], dstv)
        plsc.subcore_barrier()

        pltpu.async_copy(g_hbm.at[srcv.at[_i32(0)]], buf2.at[_i32(0)], gsem)

        def pipe(j, carry):
            b = j % 2
            pltpu.make_async_copy(g_hbm.at[srcv.at[j]], buf2.at[b],
                                  gsem).wait()

            @pl.when(j >= 1)
            def _():
                pltpu.make_async_copy(buf2.at[1 - b], acc.at[dstv.at[j - 1]],
                                      ssem).wait()

            @pl.when(j + 1 < NCH)
            def _():
                pltpu.async_copy(g_hbm.at[srcv.at[j + 1]], buf2.at[1 - b],
                                 gsem)

            pltpu.async_copy(buf2.at[b], acc.at[dstv.at[j]], ssem, add=True)
            return carry

        _fori(NCH, pipe)
        pltpu.make_async_copy(buf2.at[_i32((NCH - 1) % 2)],
                              acc.at[dstv.at[_i32(NCH - 1)]], ssem).wait()
        plsc.subcore_barrier()
        pltpu.sync_copy(acc.at[pl.ds(s * RPT, RPT)],
                        out_hbm.at[c, pl.ds(s * RPT, RPT)])

    return prop_kernel(g, src_t, dst_t)


def _tc_dis(degp):
    B = NP // 8

    def body(p_ref, dis_ref):
        deg = p_ref[0, :, 0:1] + p_ref[1, :, 0:1] + 1.0
        dis_ref[...] = lax.rsqrt(deg)

    return pl.pallas_call(
        body,
        grid=(NP // B,),
        in_specs=[pl.BlockSpec((2, B, DEGW), lambda i: (_i32(0), i, _i32(0)))],
        out_specs=pl.BlockSpec((B, 1), lambda i: (i, _i32(0))),
        out_shape=jax.ShapeDtypeStruct((NP, 1), jnp.float32),
    )(degp)


def _tc_mm1(x_tnd, W1, dis):
    B = RPT

    def body(x_ref, w_ref, dis_ref, out_ref):
        g = jnp.dot(x_ref[0], w_ref[...],
                    preferred_element_type=jnp.float32)
        out_ref[0] = g * dis_ref[...]

    return pl.pallas_call(
        body,
        grid=(T, NP // B),
        in_specs=[
            pl.BlockSpec((1, B, D), lambda t, i: (t, i, _i32(0))),
            pl.BlockSpec((D, D), lambda t, i: (_i32(0), _i32(0))),
            pl.BlockSpec((B, 1), lambda t, i: (i, _i32(0))),
        ],
        out_specs=pl.BlockSpec((1, B, D), lambda t, i: (t, i, _i32(0))),
        out_shape=jax.ShapeDtypeStruct((T, NP, D), jnp.float32),
    )(x_tnd, W1, dis)


def _tc_comb(part, g1, dis, b, W2, t):
    B = RPT

    def body(p_ref, g_ref, dis_ref, b_ref, w_ref, out_ref):
        ssum = p_ref[0] + p_ref[1] + g_ref[0]
        z = jnp.maximum(ssum * dis_ref[...] + b_ref[...], 0.0)
        h2 = jnp.dot(z, w_ref[...], preferred_element_type=jnp.float32)
        out_ref[...] = h2 * dis_ref[...]

    return pl.pallas_call(
        body,
        grid=(NP // B,),
        in_specs=[
            pl.BlockSpec((2, B, D), lambda i: (_i32(0), i, _i32(0))),
            pl.BlockSpec((1, B, D), lambda i, _t=t: (_i32(_t), i, _i32(0))),
            pl.BlockSpec((B, 1), lambda i: (i, _i32(0))),
            pl.BlockSpec((1, D), lambda i: (_i32(0), _i32(0))),
            pl.BlockSpec((D, D), lambda i: (_i32(0), _i32(0))),
        ],
        out_specs=pl.BlockSpec((B, D), lambda i: (i, _i32(0))),
        out_shape=jax.ShapeDtypeStruct((NP, D), jnp.float32),
    )(part, g1, dis, b, W2)


def _tc_final(part, g2, dis, b):
    B = RPT

    def body(p_ref, g_ref, dis_ref, b_ref, out_ref):
        ssum = p_ref[0] + p_ref[1] + g_ref[...]
        out_ref[...] = jnp.maximum(ssum * dis_ref[...] + b_ref[...], 0.0)

    return pl.pallas_call(
        body,
        grid=(NP // B,),
        in_specs=[
            pl.BlockSpec((2, B, D), lambda i: (_i32(0), i, _i32(0))),
            pl.BlockSpec((B, D), lambda i: (i, _i32(0))),
            pl.BlockSpec((B, 1), lambda i: (i, _i32(0))),
            pl.BlockSpec((1, D), lambda i: (_i32(0), _i32(0))),
        ],
        out_specs=pl.BlockSpec((B, D), lambda i: (i, _i32(0))),
        out_shape=jax.ShapeDtypeStruct((NP, D), jnp.float32),
    )(part, g2, dis, b)


def kernel(x, edge_index, W1, b1, W2, b2):
    src = edge_index[0].astype(jnp.int32)
    dst = edge_index[1].astype(jnp.int32)
    src_t = jnp.pad(src.reshape(NTILES, EPT),
                    ((0, 0), (0, EPT_PAD - EPT))).reshape(NTILES, NCH, CH)
    dst_t = jnp.pad(dst.reshape(NTILES, EPT),
                    ((0, 0), (0, EPT_PAD - EPT)),
                    constant_values=DUMP).reshape(NTILES, NCH, CH)
    x_tnd = jnp.pad(jnp.transpose(x.astype(jnp.float32), (1, 0, 2)),
                    ((0, 0), (0, NP - N), (0, 0)))
    W1 = W1.astype(jnp.float32)
    W2 = W2.astype(jnp.float32)
    b1 = b1.astype(jnp.float32).reshape(1, D)
    b2 = b2.astype(jnp.float32).reshape(1, D)

    degp = _sc_degree(dst_t)
    dis = _tc_dis(degp)
    g1 = _tc_mm1(x_tnd, W1, dis)

    outs = []
    for t in range(T):
        p1 = _sc_propagate(g1[t], src_t, dst_t)
        g2 = _tc_comb(p1, g1, dis, b1, W2, t)
        p2 = _sc_propagate(g2, src_t, dst_t)
        outs.append(_tc_final(p2, g2, dis, b2)[:N])
    return jnp.stack(outs, axis=1)
